# Initial kernel scaffold; baseline (speedup 1.0000x reference)
import jax
import jax.numpy as jnp
from jax.experimental import pallas as pl


def kernel(obs, w_node_init, w_edge_emb, w_edge_feat, w_msg_0, w_msg_1, w_msg_2, w_upd_0, w_upd_1, w_upd_2, w_pool, w_read, b_read):
    raise NotImplementedError("write your pallas kernel here")


# speedup vs baseline: 1.4803x; 1.4803x over previous
import jax
import jax.numpy as jnp
from jax.experimental import pallas as pl
from jax.experimental.pallas import tpu as pltpu


def _mpnn_kernel(adj_ref, nf_ref, nrm_ref,
                 w_init_ref, we_a_ref, we_n_ref, w_ef1_ref, w_ef2_ref,
                 w_msg_ref, w_upd_ref,
                 w_pool_ref, w_rp_ref, w_rl_ref, b_ref,
                 o_ref,
                 nfp_scr, emb_scr):
    TB, V, _ = adj_ref.shape
    n_obs = nf_ref.shape[2]
    F = w_init_ref.shape[1]
    L = w_msg_ref.shape[0]
    rows = TB * V
    f32 = jnp.float32
    relu = lambda z: jnp.maximum(z, 0.0)

    nf2d = nf_ref[...].reshape(rows, n_obs)
    inv_all = nrm_ref[...][:, :, 0].reshape(rows, 1)
    ns_all = nrm_ref[...][:, :, 1].reshape(rows, 1)

    nfp_scr[...] = jnp.dot(nf2d, we_n_ref[...], preferred_element_type=f32)

    wa3 = we_a_ref[...].reshape(1, 1, F)
    IC = 8
    for b in range(TB):
        nfp_b = nfp_scr[b * V:(b + 1) * V, :]

        def chunk(k, _, b=b, nfp_b=nfp_b):
            i0 = k * IC
            a = adj_ref[b, pl.ds(i0, IC), :]
            z = a[:, :, None] * wa3 + nfp_b[None, :, :]
            emb_scr[pl.ds(b * V + i0, IC), :] = jnp.sum(relu(z), axis=1)
            return 0

        jax.lax.fori_loop(0, V // IC, chunk, 0)

    corrs = []
    for b in range(TB):
        omm = jnp.where(adj_ref[b] == 0.0, 1.0, 0.0).astype(f32)
        rnfp = relu(nfp_scr[b * V:(b + 1) * V, :])
        corrs.append(jnp.dot(omm, rnfp, preferred_element_type=f32))
    corr = jnp.concatenate(corrs, axis=0)

    emb = (emb_scr[...] - corr) * inv_all
    edge_emb = relu(jnp.dot(emb, w_ef1_ref[...], preferred_element_type=f32)
                    + ns_all * w_ef2_ref[...])

    mu = relu(jnp.dot(nf2d, w_init_ref[...], preferred_element_type=f32))

    for l in range(L):
        aggs = []
        for b in range(TB):
            aggs.append(jnp.dot(adj_ref[b], mu[b * V:(b + 1) * V, :],
                                preferred_element_type=f32))
        agg = jnp.concatenate(aggs, axis=0) * inv_all
        msg = relu(jnp.dot(jnp.concatenate([agg, edge_emb], axis=1),
                           w_msg_ref[l], preferred_element_type=f32))
        mu = relu(jnp.dot(jnp.concatenate([mu, msg], axis=1),
                          w_upd_ref[l], preferred_element_type=f32))

    mu3 = mu.reshape(TB, V, F)
    pooled = jnp.sum(mu3, axis=1) * (1.0 / V)
    h = relu(jnp.dot(pooled, w_pool_ref[...], preferred_element_type=f32))
    s_pool = jnp.sum(h * w_rp_ref[...], axis=-1, keepdims=True)
    s_local = jnp.sum(mu * w_rl_ref[...], axis=-1, keepdims=True)
    out2d = s_local.reshape(TB, V) + s_pool + b_ref[0, 0]
    o_ref[...] = out2d[:, None, :]


def kernel(obs, w_node_init, w_edge_emb, w_edge_feat,
           w_msg_0, w_msg_1, w_msg_2,
           w_upd_0, w_upd_1, w_upd_2,
           w_pool, w_read, b_read):
    if obs.ndim == 2:
        obs = obs[None]
    obs = obs.astype(jnp.float32)
    f32 = jnp.float32
    B = obs.shape[0]
    V = obs.shape[-1]
    F = w_node_init.shape[0]
    n_obs = V

    adj = obs[:, :V]
    nf = obs[:, V:]

    norm = jnp.sum(adj != 0, axis=1).astype(f32)
    norm = jnp.where(norm == 0.0, 1.0, norm)
    nrm = jnp.stack([1.0 / norm, norm / jnp.max(norm)], axis=-1)

    wee = w_edge_emb.astype(f32)
    we_a = jnp.pad(wee[:, 0], (0, 1)).reshape(1, F)
    we_n = jnp.pad(wee[:, 1:].T, ((0, 0), (0, 1)))
    wef = w_edge_feat.astype(f32)
    w_ef1 = jnp.pad(wef[:, :F - 1].T, ((0, 1), (0, 0)))
    w_ef2 = wef[:, F - 1].reshape(1, F)
    w_init_t = w_node_init.T.astype(f32)
    w_msg = jnp.stack([w.T for w in (w_msg_0, w_msg_1, w_msg_2)]).astype(f32)
    w_upd = jnp.stack([w.T for w in (w_upd_0, w_upd_1, w_upd_2)]).astype(f32)
    w_pool_t = w_pool.T.astype(f32)
    w_r = w_read.reshape(2 * F).astype(f32)
    w_rp = w_r[:F].reshape(1, F)
    w_rl = w_r[F:].reshape(1, F)
    b_r = b_read.reshape(1, 1).astype(f32)

    TB = 4
    while B % TB:
        TB //= 2
    L = 3

    def full(shape):
        nd = len(shape)
        return pl.BlockSpec(shape, lambda i, _nd=nd: (0,) * _nd)

    out = pl.pallas_call(
        _mpnn_kernel,
        out_shape=jax.ShapeDtypeStruct((B, 1, V), f32),
        grid=(B // TB,),
        in_specs=[
            pl.BlockSpec((TB, V, V), lambda i: (i, 0, 0)),
            pl.BlockSpec((TB, V, n_obs), lambda i: (i, 0, 0)),
            pl.BlockSpec((TB, V, 2), lambda i: (i, 0, 0)),
            full((n_obs, F)), full((1, F)), full((n_obs, F)),
            full((F, F)), full((1, F)),
            full((L, 2 * F, F)), full((L, 2 * F, F)),
            full((F, F)), full((1, F)), full((1, F)), full((1, 1)),
        ],
        out_specs=pl.BlockSpec((TB, 1, V), lambda i: (i, 0, 0)),
        scratch_shapes=[
            pltpu.VMEM((TB * V, F), f32),
            pltpu.VMEM((TB * V, F), f32),
        ],
        compiler_params=pltpu.CompilerParams(
            dimension_semantics=("parallel",)),
    )(adj, nf, nrm,
      w_init_t, we_a, we_n, w_ef1, w_ef2,
      w_msg, w_upd,
      w_pool_t, w_rp, w_rl, b_r)

    return jnp.squeeze(out)


# speedup vs baseline: 1.9264x; 1.3014x over previous
import jax
import jax.numpy as jnp
from jax.experimental import pallas as pl
from jax.experimental.pallas import tpu as pltpu


def _mpnn_kernel(adj_ref, nf_ref, nrm_ref,
                 w_init_ref, we_a_ref, we_n_ref, w_ef1_ref, w_ef2_ref,
                 w_msg_ref, w_upd_ref,
                 w_pool_ref, w_rp_ref, w_rl_ref, b_ref,
                 o_ref):
    TB, V, _ = adj_ref.shape
    n_obs = nf_ref.shape[2]
    F = w_init_ref.shape[1]
    L = w_msg_ref.shape[0]
    rows = TB * V
    f32 = jnp.float32
    relu = lambda z: jnp.maximum(z, 0.0)

    nf2d = nf_ref[...].reshape(rows, n_obs)
    inv_all = nrm_ref[...][:, :, 0].reshape(rows, 1)
    ns_all = nrm_ref[...][:, :, 1].reshape(rows, 1)

    nfp = jnp.dot(nf2d, we_n_ref[...], preferred_element_type=f32)

    wa3 = we_a_ref[...].reshape(1, 1, F)
    IC = 16
    s_chunks = []
    for b in range(TB):
        nfp_b = nfp[b * V:(b + 1) * V, :][None, :, :]
        for k in range(V // IC):
            a = adj_ref[b, k * IC:(k + 1) * IC, :]
            z = a[:, :, None] * wa3 + nfp_b
            s_chunks.append(jnp.sum(relu(z), axis=1))
    s_full = jnp.concatenate(s_chunks, axis=0)

    corrs = []
    for b in range(TB):
        omm = jnp.where(adj_ref[b] == 0.0, 1.0, 0.0).astype(f32)
        rnfp = relu(nfp[b * V:(b + 1) * V, :])
        corrs.append(jnp.dot(omm, rnfp, preferred_element_type=f32))
    corr = jnp.concatenate(corrs, axis=0)

    emb = (s_full - corr) * inv_all
    edge_emb = relu(jnp.dot(emb, w_ef1_ref[...], preferred_element_type=f32)
                    + ns_all * w_ef2_ref[...])

    mu = relu(jnp.dot(nf2d, w_init_ref[...], preferred_element_type=f32))

    for l in range(L):
        aggs = []
        for b in range(TB):
            aggs.append(jnp.dot(adj_ref[b], mu[b * V:(b + 1) * V, :],
                                preferred_element_type=f32))
        agg = jnp.concatenate(aggs, axis=0) * inv_all
        msg = relu(jnp.dot(jnp.concatenate([agg, edge_emb], axis=1),
                           w_msg_ref[l], preferred_element_type=f32))
        mu = relu(jnp.dot(jnp.concatenate([mu, msg], axis=1),
                          w_upd_ref[l], preferred_element_type=f32))

    mu3 = mu.reshape(TB, V, F)
    pooled = jnp.sum(mu3, axis=1) * (1.0 / V)
    h = relu(jnp.dot(pooled, w_pool_ref[...], preferred_element_type=f32))
    s_pool = jnp.sum(h * w_rp_ref[...], axis=-1, keepdims=True)
    s_local = jnp.sum(mu * w_rl_ref[...], axis=-1, keepdims=True)
    out2d = s_local.reshape(TB, V) + s_pool + b_ref[0, 0]
    o_ref[...] = out2d[:, None, :]


def kernel(obs, w_node_init, w_edge_emb, w_edge_feat,
           w_msg_0, w_msg_1, w_msg_2,
           w_upd_0, w_upd_1, w_upd_2,
           w_pool, w_read, b_read):
    if obs.ndim == 2:
        obs = obs[None]
    obs = obs.astype(jnp.float32)
    f32 = jnp.float32
    B = obs.shape[0]
    V = obs.shape[-1]
    F = w_node_init.shape[0]
    n_obs = V

    adj = obs[:, :V]
    nf = obs[:, V:]

    norm = jnp.sum(adj != 0, axis=1).astype(f32)
    norm = jnp.where(norm == 0.0, 1.0, norm)
    nrm = jnp.stack([1.0 / norm, norm / jnp.max(norm)], axis=-1)

    wee = w_edge_emb.astype(f32)
    we_a = jnp.pad(wee[:, 0], (0, 1)).reshape(1, F)
    we_n = jnp.pad(wee[:, 1:].T, ((0, 0), (0, 1)))
    wef = w_edge_feat.astype(f32)
    w_ef1 = jnp.pad(wef[:, :F - 1].T, ((0, 1), (0, 0)))
    w_ef2 = wef[:, F - 1].reshape(1, F)
    w_init_t = w_node_init.T.astype(f32)
    w_msg = jnp.stack([w.T for w in (w_msg_0, w_msg_1, w_msg_2)]).astype(f32)
    w_upd = jnp.stack([w.T for w in (w_upd_0, w_upd_1, w_upd_2)]).astype(f32)
    w_pool_t = w_pool.T.astype(f32)
    w_r = w_read.reshape(2 * F).astype(f32)
    w_rp = w_r[:F].reshape(1, F)
    w_rl = w_r[F:].reshape(1, F)
    b_r = b_read.reshape(1, 1).astype(f32)

    TB = 2
    while B % TB:
        TB //= 2
    L = 3

    def full(shape):
        nd = len(shape)
        return pl.BlockSpec(shape, lambda i, _nd=nd: (0,) * _nd)

    out = pl.pallas_call(
        _mpnn_kernel,
        out_shape=jax.ShapeDtypeStruct((B, 1, V), f32),
        grid=(B // TB,),
        in_specs=[
            pl.BlockSpec((TB, V, V), lambda i: (i, 0, 0)),
            pl.BlockSpec((TB, V, n_obs), lambda i: (i, 0, 0)),
            pl.BlockSpec((TB, V, 2), lambda i: (i, 0, 0)),
            full((n_obs, F)), full((1, F)), full((n_obs, F)),
            full((F, F)), full((1, F)),
            full((L, 2 * F, F)), full((L, 2 * F, F)),
            full((F, F)), full((1, F)), full((1, F)), full((1, 1)),
        ],
        out_specs=pl.BlockSpec((TB, 1, V), lambda i: (i, 0, 0)),
        compiler_params=pltpu.CompilerParams(
            dimension_semantics=("parallel",)),
    )(adj, nf, nrm,
      w_init_t, we_a, we_n, w_ef1, w_ef2,
      w_msg, w_upd,
      w_pool_t, w_rp, w_rl, b_r)

    return jnp.squeeze(out)


# speedup vs baseline: 2.3046x; 1.1963x over previous
import jax
import jax.numpy as jnp
from jax.experimental import pallas as pl
from jax.experimental.pallas import tpu as pltpu


def _mpnn_kernel(adj_ref, nf_ref, nrm_ref,
                 w_init_ref, we_a_ref, we_n_ref, w_ef1_ref, w_ef2_ref,
                 w_msg_ref, w_upd_ref,
                 w_pool_ref, w_rp_ref, w_rl_ref, b_ref,
                 o_ref):
    TB, V, _ = adj_ref.shape
    n_obs = nf_ref.shape[2]
    F = w_init_ref.shape[1]
    L = w_msg_ref.shape[0]
    rows = TB * V
    f32 = jnp.float32
    relu = lambda z: jnp.maximum(z, 0.0)

    nf2d = nf_ref[...].reshape(rows, n_obs)
    inv_all = nrm_ref[...][:, :, 0].reshape(rows, 1)
    ns_all = nrm_ref[...][:, :, 1].reshape(rows, 1)

    nfp = jnp.dot(nf2d, we_n_ref[...], preferred_element_type=f32)

    wa3 = we_a_ref[...].reshape(1, 1, F)
    IC = 16
    s_chunks = []
    for b in range(TB):
        nfp_b = nfp[b * V:(b + 1) * V, :][None, :, :]
        for k in range(V // IC):
            a = adj_ref[b, k * IC:(k + 1) * IC, :]
            z = a[:, :, None] * wa3 + nfp_b
            s_chunks.append(jnp.sum(relu(z), axis=1))
    s_full = jnp.concatenate(s_chunks, axis=0)

    corrs = []
    for b in range(TB):
        omm = jnp.where(adj_ref[b] == 0.0, 1.0, 0.0).astype(f32)
        rnfp = relu(nfp[b * V:(b + 1) * V, :])
        corrs.append(jnp.dot(omm, rnfp, preferred_element_type=f32))
    corr = jnp.concatenate(corrs, axis=0)

    emb = (s_full - corr) * inv_all
    edge_emb = relu(jnp.dot(emb, w_ef1_ref[...], preferred_element_type=f32)
                    + ns_all * w_ef2_ref[...])

    mu = relu(jnp.dot(nf2d, w_init_ref[...], preferred_element_type=f32))

    for l in range(L):
        aggs = []
        for b in range(TB):
            aggs.append(jnp.dot(adj_ref[b], mu[b * V:(b + 1) * V, :],
                                preferred_element_type=f32))
        agg = jnp.concatenate(aggs, axis=0) * inv_all
        msg = relu(jnp.dot(jnp.concatenate([agg, edge_emb], axis=1),
                           w_msg_ref[l], preferred_element_type=f32))
        mu = relu(jnp.dot(jnp.concatenate([mu, msg], axis=1),
                          w_upd_ref[l], preferred_element_type=f32))

    mu3 = mu.reshape(TB, V, F)
    pooled = jnp.sum(mu3, axis=1) * (1.0 / V)
    h = relu(jnp.dot(pooled, w_pool_ref[...], preferred_element_type=f32))
    s_pool = jnp.sum(h * w_rp_ref[...], axis=-1, keepdims=True)
    s_local = jnp.sum(mu * w_rl_ref[...], axis=-1, keepdims=True)
    out2d = s_local.reshape(TB, V) + s_pool + b_ref[0, 0]
    o_ref[...] = out2d[:, None, :]


def kernel(obs, w_node_init, w_edge_emb, w_edge_feat,
           w_msg_0, w_msg_1, w_msg_2,
           w_upd_0, w_upd_1, w_upd_2,
           w_pool, w_read, b_read):
    if obs.ndim == 2:
        obs = obs[None]
    obs = obs.astype(jnp.float32)
    f32 = jnp.float32
    B = obs.shape[0]
    V = obs.shape[-1]
    F = w_node_init.shape[0]
    n_obs = V

    adj = obs[:, :V]
    nf = obs[:, V:]

    norm = jnp.sum(adj != 0, axis=1).astype(f32)
    norm = jnp.where(norm == 0.0, 1.0, norm)
    nrm = jnp.stack([1.0 / norm, norm / jnp.max(norm)], axis=-1)

    wee = w_edge_emb.astype(f32)
    we_a = jnp.pad(wee[:, 0], (0, 1)).reshape(1, F)
    we_n = jnp.pad(wee[:, 1:].T, ((0, 0), (0, 1)))
    wef = w_edge_feat.astype(f32)
    w_ef1 = jnp.pad(wef[:, :F - 1].T, ((0, 1), (0, 0)))
    w_ef2 = wef[:, F - 1].reshape(1, F)
    w_init_t = w_node_init.T.astype(f32)
    w_msg = jnp.stack([w.T for w in (w_msg_0, w_msg_1, w_msg_2)]).astype(f32)
    w_upd = jnp.stack([w.T for w in (w_upd_0, w_upd_1, w_upd_2)]).astype(f32)
    w_pool_t = w_pool.T.astype(f32)
    w_r = w_read.reshape(2 * F).astype(f32)
    w_rp = w_r[:F].reshape(1, F)
    w_rl = w_r[F:].reshape(1, F)
    b_r = b_read.reshape(1, 1).astype(f32)

    TB = 8
    while B % TB:
        TB //= 2
    L = 3

    def full(shape):
        nd = len(shape)
        return pl.BlockSpec(shape, lambda i, _nd=nd: (0,) * _nd)

    out = pl.pallas_call(
        _mpnn_kernel,
        out_shape=jax.ShapeDtypeStruct((B, 1, V), f32),
        grid=(B // TB,),
        in_specs=[
            pl.BlockSpec((TB, V, V), lambda i: (i, 0, 0)),
            pl.BlockSpec((TB, V, n_obs), lambda i: (i, 0, 0)),
            pl.BlockSpec((TB, V, 2), lambda i: (i, 0, 0)),
            full((n_obs, F)), full((1, F)), full((n_obs, F)),
            full((F, F)), full((1, F)),
            full((L, 2 * F, F)), full((L, 2 * F, F)),
            full((F, F)), full((1, F)), full((1, F)), full((1, 1)),
        ],
        out_specs=pl.BlockSpec((TB, 1, V), lambda i: (i, 0, 0)),
        compiler_params=pltpu.CompilerParams(
            dimension_semantics=("parallel",)),
    )(adj, nf, nrm,
      w_init_t, we_a, we_n, w_ef1, w_ef2,
      w_msg, w_upd,
      w_pool_t, w_rp, w_rl, b_r)

    return jnp.squeeze(out)


# speedup vs baseline: 2.4025x; 1.0425x over previous
import jax
import jax.numpy as jnp
from jax.experimental import pallas as pl
from jax.experimental.pallas import tpu as pltpu


def _mpnn_kernel(obs_ref,
                 w_init_ref, we_a_ref, we_n_ref, w_ef1_ref, w_ef2s_ref,
                 w_msg_ref, w_upd_ref,
                 w_pool_ref, w_rp_ref, w_rl_ref, b_ref,
                 o_ref):
    TB = obs_ref.shape[0]
    V = obs_ref.shape[2]
    n_obs = V
    F = w_init_ref.shape[1]
    L = w_msg_ref.shape[0]
    rows = TB * V
    f32 = jnp.float32
    relu = lambda z: jnp.maximum(z, 0.0)

    nf2d = obs_ref[:, V:, :].reshape(rows, n_obs)

    nfp = jnp.dot(nf2d, we_n_ref[...], preferred_element_type=f32)

    wa3 = we_a_ref[...].reshape(1, 1, F)
    IC = 16
    s_chunks = []
    for b in range(TB):
        nfp_b = nfp[b * V:(b + 1) * V, :][None, :, :]
        for k in range(V // IC):
            a = obs_ref[b, k * IC:(k + 1) * IC, :]
            z = a[:, :, None] * wa3 + nfp_b
            s_chunks.append(jnp.sum(relu(z), axis=1))
    s_full = jnp.concatenate(s_chunks, axis=0)

    ones_col = jnp.ones((V, 1), f32)
    dn = (((0,), (0,)), ((), ()))
    corrs, invs, nrms = [], [], []
    for b in range(TB):
        adj_b = obs_ref[b, :V, :]
        omm = jnp.where(adj_b == 0.0, 1.0, 0.0).astype(f32)
        rnfp = relu(nfp[b * V:(b + 1) * V, :])
        corrs.append(jnp.dot(omm, rnfp, preferred_element_type=f32))
        zc = jax.lax.dot_general(omm, ones_col, dn,
                                 preferred_element_type=f32)
        nrm = jnp.maximum(V * 1.0 - zc, 1.0)
        nrms.append(nrm)
        invs.append(1.0 / nrm)
    corr = jnp.concatenate(corrs, axis=0)
    inv_all = jnp.concatenate(invs, axis=0)
    nrm_all = jnp.concatenate(nrms, axis=0)

    emb = (s_full - corr) * inv_all
    edge_emb = relu(jnp.dot(emb, w_ef1_ref[...], preferred_element_type=f32)
                    + nrm_all * w_ef2s_ref[...])

    mu = relu(jnp.dot(nf2d, w_init_ref[...], preferred_element_type=f32))

    for l in range(L):
        aggs = []
        for b in range(TB):
            aggs.append(jnp.dot(obs_ref[b, :V, :], mu[b * V:(b + 1) * V, :],
                                preferred_element_type=f32))
        agg = jnp.concatenate(aggs, axis=0) * inv_all
        msg = relu(jnp.dot(jnp.concatenate([agg, edge_emb], axis=1),
                           w_msg_ref[l], preferred_element_type=f32))
        mu = relu(jnp.dot(jnp.concatenate([mu, msg], axis=1),
                          w_upd_ref[l], preferred_element_type=f32))

    mu3 = mu.reshape(TB, V, F)
    pooled = jnp.sum(mu3, axis=1) * (1.0 / V)
    h = relu(jnp.dot(pooled, w_pool_ref[...], preferred_element_type=f32))
    s_pool = jnp.sum(h * w_rp_ref[...], axis=-1, keepdims=True)
    s_local = jnp.sum(mu * w_rl_ref[...], axis=-1, keepdims=True)
    out2d = s_local.reshape(TB, V) + s_pool + b_ref[0, 0]
    o_ref[...] = out2d[:, None, :]


def kernel(obs, w_node_init, w_edge_emb, w_edge_feat,
           w_msg_0, w_msg_1, w_msg_2,
           w_upd_0, w_upd_1, w_upd_2,
           w_pool, w_read, b_read):
    if obs.ndim == 2:
        obs = obs[None]
    obs = obs.astype(jnp.float32)
    f32 = jnp.float32
    B = obs.shape[0]
    V = obs.shape[-1]
    F = w_node_init.shape[0]
    n_obs = V

    norm = jnp.sum(obs[:, :V] != 0, axis=1).astype(f32)
    maxn = jnp.max(jnp.where(norm == 0.0, 1.0, norm))

    wee = w_edge_emb.astype(f32)
    we_a = jnp.pad(wee[:, 0], (0, 1)).reshape(1, F)
    we_n = jnp.pad(wee[:, 1:].T, ((0, 0), (0, 1)))
    wef = w_edge_feat.astype(f32)
    w_ef1 = jnp.pad(wef[:, :F - 1].T, ((0, 1), (0, 0)))
    w_ef2s = wef[:, F - 1].reshape(1, F) / maxn
    w_init_t = w_node_init.T.astype(f32)
    w_msg = jnp.stack([w.T for w in (w_msg_0, w_msg_1, w_msg_2)]).astype(f32)
    w_upd = jnp.stack([w.T for w in (w_upd_0, w_upd_1, w_upd_2)]).astype(f32)
    w_pool_t = w_pool.T.astype(f32)
    w_r = w_read.reshape(2 * F).astype(f32)
    w_rp = w_r[:F].reshape(1, F)
    w_rl = w_r[F:].reshape(1, F)
    b_r = b_read.reshape(1, 1).astype(f32)

    TB = 8
    while B % TB:
        TB //= 2
    L = 3

    def full(shape):
        nd = len(shape)
        return pl.BlockSpec(shape, lambda i, _nd=nd: (0,) * _nd)

    out = pl.pallas_call(
        _mpnn_kernel,
        out_shape=jax.ShapeDtypeStruct((B, 1, V), f32),
        grid=(B // TB,),
        in_specs=[
            pl.BlockSpec((TB, 2 * V, V), lambda i: (i, 0, 0)),
            full((n_obs, F)), full((1, F)), full((n_obs, F)),
            full((F, F)), full((1, F)),
            full((L, 2 * F, F)), full((L, 2 * F, F)),
            full((F, F)), full((1, F)), full((1, F)), full((1, 1)),
        ],
        out_specs=pl.BlockSpec((TB, 1, V), lambda i: (i, 0, 0)),
        compiler_params=pltpu.CompilerParams(
            dimension_semantics=("parallel",)),
    )(obs,
      w_init_t, we_a, we_n, w_ef1, w_ef2s,
      w_msg, w_upd,
      w_pool_t, w_rp, w_rl, b_r)

    return jnp.squeeze(out)


# speedup vs baseline: 4.0841x; 1.7000x over previous
import jax
import jax.numpy as jnp
from jax.experimental import pallas as pl
from jax.experimental.pallas import tpu as pltpu


def _mpnn_kernel(obs_ref,
                 w_init_ref, we_a_ref, we_n_ref, w_ef1_ref, w_ef2s_ref,
                 w_msg_ref, w_upd_ref,
                 w_pool_ref, w_rp_ref, w_rl_ref, b_ref,
                 o_ref):
    TB = obs_ref.shape[0]
    V = obs_ref.shape[2]
    n_obs = V
    F = w_init_ref.shape[1]
    L = w_msg_ref.shape[0]
    rows = TB * V
    f32 = jnp.float32
    relu = lambda z: jnp.maximum(z, 0.0)

    nf2d = obs_ref[:, V:, :].reshape(rows, n_obs)

    nfp = jnp.dot(nf2d, we_n_ref[...], preferred_element_type=f32)

    bf16 = jnp.bfloat16
    wa3 = we_a_ref[...].astype(bf16).reshape(1, 1, F)
    IC = 16
    s_chunks = []
    for b in range(TB):
        nfp_b = nfp[b * V:(b + 1) * V, :].astype(bf16)[None, :, :]
        a_bf = obs_ref[b, :V, :].astype(bf16)
        for k in range(V // IC):
            a = a_bf[k * IC:(k + 1) * IC, :]
            z = a[:, :, None] * wa3 + nfp_b
            r = relu(z)
            t = r[:, :V // 2, :] + r[:, V // 2:, :]
            t = t[:, :V // 4, :] + t[:, V // 4:, :]
            t = t[:, :V // 8, :] + t[:, V // 8:, :]
            tf = t.astype(f32)
            s_chunks.append(jnp.sum(tf, axis=1))
    s_full = jnp.concatenate(s_chunks, axis=0)

    ones_col = jnp.ones((V, 1), f32)
    dn = (((0,), (0,)), ((), ()))
    corrs, invs, nrms = [], [], []
    for b in range(TB):
        adj_b = obs_ref[b, :V, :]
        omm = jnp.where(adj_b == 0.0, 1.0, 0.0).astype(f32)
        rnfp = relu(nfp[b * V:(b + 1) * V, :])
        corrs.append(jnp.dot(omm, rnfp, preferred_element_type=f32))
        zc = jax.lax.dot_general(omm, ones_col, dn,
                                 preferred_element_type=f32)
        nrm = jnp.maximum(V * 1.0 - zc, 1.0)
        nrms.append(nrm)
        invs.append(1.0 / nrm)
    corr = jnp.concatenate(corrs, axis=0)
    inv_all = jnp.concatenate(invs, axis=0)
    nrm_all = jnp.concatenate(nrms, axis=0)

    emb = (s_full - corr) * inv_all
    edge_emb = relu(jnp.dot(emb, w_ef1_ref[...], preferred_element_type=f32)
                    + nrm_all * w_ef2s_ref[...])

    mu = relu(jnp.dot(nf2d, w_init_ref[...], preferred_element_type=f32))

    for l in range(L):
        aggs = []
        for b in range(TB):
            aggs.append(jnp.dot(obs_ref[b, :V, :], mu[b * V:(b + 1) * V, :],
                                preferred_element_type=f32))
        agg = jnp.concatenate(aggs, axis=0) * inv_all
        msg = relu(jnp.dot(jnp.concatenate([agg, edge_emb], axis=1),
                           w_msg_ref[l], preferred_element_type=f32))
        mu = relu(jnp.dot(jnp.concatenate([mu, msg], axis=1),
                          w_upd_ref[l], preferred_element_type=f32))

    mu3 = mu.reshape(TB, V, F)
    pooled = jnp.sum(mu3, axis=1) * (1.0 / V)
    h = relu(jnp.dot(pooled, w_pool_ref[...], preferred_element_type=f32))
    s_pool = jnp.sum(h * w_rp_ref[...], axis=-1, keepdims=True)
    s_local = jnp.sum(mu * w_rl_ref[...], axis=-1, keepdims=True)
    out2d = s_local.reshape(TB, V) + s_pool + b_ref[0, 0]
    o_ref[...] = out2d[:, None, :]


def kernel(obs, w_node_init, w_edge_emb, w_edge_feat,
           w_msg_0, w_msg_1, w_msg_2,
           w_upd_0, w_upd_1, w_upd_2,
           w_pool, w_read, b_read):
    if obs.ndim == 2:
        obs = obs[None]
    obs = obs.astype(jnp.float32)
    f32 = jnp.float32
    B = obs.shape[0]
    V = obs.shape[-1]
    F = w_node_init.shape[0]
    n_obs = V

    norm = jnp.sum(obs[:, :V] != 0, axis=1).astype(f32)
    maxn = jnp.max(jnp.where(norm == 0.0, 1.0, norm))

    wee = w_edge_emb.astype(f32)
    we_a = jnp.pad(wee[:, 0], (0, 1)).reshape(1, F)
    we_n = jnp.pad(wee[:, 1:].T, ((0, 0), (0, 1)))
    wef = w_edge_feat.astype(f32)
    w_ef1 = jnp.pad(wef[:, :F - 1].T, ((0, 1), (0, 0)))
    w_ef2s = wef[:, F - 1].reshape(1, F) / maxn
    w_init_t = w_node_init.T.astype(f32)
    w_msg = jnp.stack([w.T for w in (w_msg_0, w_msg_1, w_msg_2)]).astype(f32)
    w_upd = jnp.stack([w.T for w in (w_upd_0, w_upd_1, w_upd_2)]).astype(f32)
    w_pool_t = w_pool.T.astype(f32)
    w_r = w_read.reshape(2 * F).astype(f32)
    w_rp = w_r[:F].reshape(1, F)
    w_rl = w_r[F:].reshape(1, F)
    b_r = b_read.reshape(1, 1).astype(f32)

    TB = 8
    while B % TB:
        TB //= 2
    L = 3

    def full(shape):
        nd = len(shape)
        return pl.BlockSpec(shape, lambda i, _nd=nd: (0,) * _nd)

    out = pl.pallas_call(
        _mpnn_kernel,
        out_shape=jax.ShapeDtypeStruct((B, 1, V), f32),
        grid=(B // TB,),
        in_specs=[
            pl.BlockSpec((TB, 2 * V, V), lambda i: (i, 0, 0)),
            full((n_obs, F)), full((1, F)), full((n_obs, F)),
            full((F, F)), full((1, F)),
            full((L, 2 * F, F)), full((L, 2 * F, F)),
            full((F, F)), full((1, F)), full((1, F)), full((1, 1)),
        ],
        out_specs=pl.BlockSpec((TB, 1, V), lambda i: (i, 0, 0)),
        compiler_params=pltpu.CompilerParams(
            dimension_semantics=("parallel",)),
    )(obs,
      w_init_t, we_a, we_n, w_ef1, w_ef2s,
      w_msg, w_upd,
      w_pool_t, w_rp, w_rl, b_r)

    return jnp.squeeze(out)


# speedup vs baseline: 4.3162x; 1.0568x over previous
import jax
import jax.numpy as jnp
from jax.experimental import pallas as pl
from jax.experimental.pallas import tpu as pltpu


def _mpnn_kernel(obs_ref,
                 w_init_ref, we_a_ref, we_n_ref, w_ef1_ref, w_ef2s_ref,
                 w_msg_ref, w_upd_ref,
                 w_pool_ref, w_rp_ref, w_rl_ref, b_ref,
                 o_ref):
    TB = obs_ref.shape[0]
    V = obs_ref.shape[2]
    n_obs = V
    F = w_init_ref.shape[1]
    L = w_msg_ref.shape[0]
    rows = TB * V
    f32 = jnp.float32
    relu = lambda z: jnp.maximum(z, 0.0)

    nf2d = obs_ref[:, V:, :].reshape(rows, n_obs)

    nfp = jnp.dot(nf2d, we_n_ref[...], preferred_element_type=f32)

    bf16 = jnp.bfloat16
    wa3 = we_a_ref[...].astype(bf16).reshape(1, 1, F)
    IC = 16
    s_chunks = []
    for b in range(TB):
        nfp_b = nfp[b * V:(b + 1) * V, :].astype(bf16)[None, :, :]
        a_bf = obs_ref[b, :V, :].astype(bf16)
        for k in range(V // IC):
            a = a_bf[k * IC:(k + 1) * IC, :]
            z = a[:, :, None] * wa3 + nfp_b
            r = relu(z)
            t = r[:, :V // 2, :] + r[:, V // 2:, :]
            t = t[:, :V // 4, :] + t[:, V // 4:, :]
            t = t[:, :V // 8, :] + t[:, V // 8:, :]
            tf = t.astype(f32)
            s_chunks.append(jnp.sum(tf, axis=1))
    s_full = jnp.concatenate(s_chunks, axis=0)

    ones_col = jnp.ones((V, 1), f32)
    dn = (((0,), (0,)), ((), ()))
    corrs, invs, nrms = [], [], []
    for b in range(TB):
        adj_b = obs_ref[b, :V, :]
        omm = jnp.where(adj_b == 0.0, 1.0, 0.0).astype(f32)
        rnfp = relu(nfp[b * V:(b + 1) * V, :])
        corrs.append(jnp.dot(omm, rnfp, preferred_element_type=f32))
        zc = jax.lax.dot_general(omm, ones_col, dn,
                                 preferred_element_type=f32)
        nrm = jnp.maximum(V * 1.0 - zc, 1.0)
        nrms.append(nrm)
        invs.append(1.0 / nrm)
    corr = jnp.concatenate(corrs, axis=0)
    inv_all = jnp.concatenate(invs, axis=0)
    nrm_all = jnp.concatenate(nrms, axis=0)

    emb = (s_full - corr) * inv_all
    edge_emb = relu(jnp.dot(emb, w_ef1_ref[...], preferred_element_type=f32)
                    + nrm_all * w_ef2s_ref[...])

    mu = relu(jnp.dot(nf2d, w_init_ref[...], preferred_element_type=f32))

    for l in range(L):
        aggs = []
        for b in range(TB):
            aggs.append(jnp.dot(obs_ref[b, :V, :], mu[b * V:(b + 1) * V, :],
                                preferred_element_type=f32))
        agg = jnp.concatenate(aggs, axis=0) * inv_all
        msg = relu(jnp.dot(jnp.concatenate([agg, edge_emb], axis=1),
                           w_msg_ref[l], preferred_element_type=f32))
        mu = relu(jnp.dot(jnp.concatenate([mu, msg], axis=1),
                          w_upd_ref[l], preferred_element_type=f32))

    mu3 = mu.reshape(TB, V, F)
    pooled = jnp.sum(mu3, axis=1) * (1.0 / V)
    h = relu(jnp.dot(pooled, w_pool_ref[...], preferred_element_type=f32))
    s_pool = jnp.sum(h * w_rp_ref[...], axis=-1, keepdims=True)
    s_local = jnp.sum(mu * w_rl_ref[...], axis=-1, keepdims=True)
    out2d = s_local.reshape(TB, V) + s_pool + b_ref[0, 0]
    o_ref[...] = out2d[:, None, :]


def kernel(obs, w_node_init, w_edge_emb, w_edge_feat,
           w_msg_0, w_msg_1, w_msg_2,
           w_upd_0, w_upd_1, w_upd_2,
           w_pool, w_read, b_read):
    if obs.ndim == 2:
        obs = obs[None]
    obs = obs.astype(jnp.float32)
    f32 = jnp.float32
    B = obs.shape[0]
    V = obs.shape[-1]
    F = w_node_init.shape[0]
    n_obs = V

    norm = jnp.sum(obs[:, :V] != 0, axis=1).astype(f32)
    maxn = jnp.max(jnp.where(norm == 0.0, 1.0, norm))

    wee = w_edge_emb.astype(f32)
    we_a = jnp.pad(wee[:, 0], (0, 1)).reshape(1, F)
    we_n = jnp.pad(wee[:, 1:].T, ((0, 0), (0, 1)))
    wef = w_edge_feat.astype(f32)
    w_ef1 = jnp.pad(wef[:, :F - 1].T, ((0, 1), (0, 0)))
    w_ef2s = wef[:, F - 1].reshape(1, F) / maxn
    w_init_t = w_node_init.T.astype(f32)
    w_msg = jnp.stack([w.T for w in (w_msg_0, w_msg_1, w_msg_2)]).astype(f32)
    w_upd = jnp.stack([w.T for w in (w_upd_0, w_upd_1, w_upd_2)]).astype(f32)
    w_pool_t = w_pool.T.astype(f32)
    w_r = w_read.reshape(2 * F).astype(f32)
    w_rp = w_r[:F].reshape(1, F)
    w_rl = w_r[F:].reshape(1, F)
    b_r = b_read.reshape(1, 1).astype(f32)

    TB = 16
    while B % TB:
        TB //= 2
    L = 3

    def full(shape):
        nd = len(shape)
        return pl.BlockSpec(shape, lambda i, _nd=nd: (0,) * _nd)

    out = pl.pallas_call(
        _mpnn_kernel,
        out_shape=jax.ShapeDtypeStruct((B, 1, V), f32),
        grid=(B // TB,),
        in_specs=[
            pl.BlockSpec((TB, 2 * V, V), lambda i: (i, 0, 0)),
            full((n_obs, F)), full((1, F)), full((n_obs, F)),
            full((F, F)), full((1, F)),
            full((L, 2 * F, F)), full((L, 2 * F, F)),
            full((F, F)), full((1, F)), full((1, F)), full((1, 1)),
        ],
        out_specs=pl.BlockSpec((TB, 1, V), lambda i: (i, 0, 0)),
        compiler_params=pltpu.CompilerParams(
            dimension_semantics=("parallel",)),
    )(obs,
      w_init_t, we_a, we_n, w_ef1, w_ef2s,
      w_msg, w_upd,
      w_pool_t, w_rp, w_rl, b_r)

    return jnp.squeeze(out)


# speedup vs baseline: 4.3401x; 1.0055x over previous
import jax
import jax.numpy as jnp
from jax.experimental import pallas as pl
from jax.experimental.pallas import tpu as pltpu


def _mpnn_kernel(obs_ref,
                 w_init_ref, we_a_ref, we_n_ref, w_ef1_ref, w_ef2s_ref,
                 w_msg_ref, w_upd_ref,
                 w_pool_ref, w_rp_ref, w_rl_ref, b_ref,
                 o_ref):
    TB = obs_ref.shape[0]
    V = obs_ref.shape[2]
    n_obs = V
    F = w_init_ref.shape[1]
    L = w_msg_ref.shape[0]
    rows = TB * V
    f32 = jnp.float32
    relu = lambda z: jnp.maximum(z, 0.0)

    nf2d = obs_ref[:, V:, :].reshape(rows, n_obs)

    nfp = jnp.dot(nf2d, we_n_ref[...], preferred_element_type=f32)

    ones_col = jnp.ones((V, 1), f32)
    dn = (((0,), (0,)), ((), ()))
    corrs, invs, nrms = [], [], []
    for b in range(TB):
        adj_b = obs_ref[b, :V, :]
        omm = jnp.where(adj_b == 0.0, 1.0, 0.0).astype(f32)
        rnfp = relu(nfp[b * V:(b + 1) * V, :])
        corrs.append(jnp.dot(omm, rnfp, preferred_element_type=f32))
        zc = jax.lax.dot_general(omm, ones_col, dn,
                                 preferred_element_type=f32)
        nrm = jnp.maximum(V * 1.0 - zc, 1.0)
        nrms.append(nrm)
        invs.append(1.0 / nrm)
    corr = jnp.concatenate(corrs, axis=0)
    inv_all = jnp.concatenate(invs, axis=0)
    nrm_all = jnp.concatenate(nrms, axis=0)

    mu = relu(jnp.dot(nf2d, w_init_ref[...], preferred_element_type=f32))

    bf16 = jnp.bfloat16
    wa3 = we_a_ref[...].astype(bf16).reshape(1, 1, F)
    IC = 16
    s_chunks = []
    for b in range(TB):
        nfp_b = nfp[b * V:(b + 1) * V, :].astype(bf16)[None, :, :]
        a_bf = obs_ref[b, :V, :].astype(bf16)
        for k in range(V // IC):
            a = a_bf[k * IC:(k + 1) * IC, :]
            z = a[:, :, None] * wa3 + nfp_b
            r = relu(z)
            t = r[:, :V // 2, :] + r[:, V // 2:, :]
            t = t[:, :V // 4, :] + t[:, V // 4:, :]
            t = t[:, :V // 8, :] + t[:, V // 8:, :]
            tf = t.astype(f32)
            s_chunks.append(jnp.sum(tf, axis=1))
    s_full = jnp.concatenate(s_chunks, axis=0)

    emb = (s_full - corr) * inv_all
    edge_emb = relu(jnp.dot(emb, w_ef1_ref[...], preferred_element_type=f32)
                    + nrm_all * w_ef2s_ref[...])

    for l in range(L):
        aggs = []
        for b in range(TB):
            aggs.append(jnp.dot(obs_ref[b, :V, :], mu[b * V:(b + 1) * V, :],
                                preferred_element_type=f32))
        agg = jnp.concatenate(aggs, axis=0) * inv_all
        msg = relu(jnp.dot(jnp.concatenate([agg, edge_emb], axis=1),
                           w_msg_ref[l], preferred_element_type=f32))
        mu = relu(jnp.dot(jnp.concatenate([mu, msg], axis=1),
                          w_upd_ref[l], preferred_element_type=f32))

    mu3 = mu.reshape(TB, V, F)
    pooled = jnp.sum(mu3, axis=1) * (1.0 / V)
    h = relu(jnp.dot(pooled, w_pool_ref[...], preferred_element_type=f32))
    s_pool = jnp.sum(h * w_rp_ref[...], axis=-1, keepdims=True)
    s_local = jnp.sum(mu * w_rl_ref[...], axis=-1, keepdims=True)
    out2d = s_local.reshape(TB, V) + s_pool + b_ref[0, 0]
    o_ref[...] = out2d[:, None, :]


def kernel(obs, w_node_init, w_edge_emb, w_edge_feat,
           w_msg_0, w_msg_1, w_msg_2,
           w_upd_0, w_upd_1, w_upd_2,
           w_pool, w_read, b_read):
    if obs.ndim == 2:
        obs = obs[None]
    obs = obs.astype(jnp.float32)
    f32 = jnp.float32
    B = obs.shape[0]
    V = obs.shape[-1]
    F = w_node_init.shape[0]
    n_obs = V

    norm = jnp.sum(obs[:, :V] != 0, axis=1).astype(f32)
    maxn = jnp.max(jnp.where(norm == 0.0, 1.0, norm))

    wee = w_edge_emb.astype(f32)
    we_a = jnp.pad(wee[:, 0], (0, 1)).reshape(1, F)
    we_n = jnp.pad(wee[:, 1:].T, ((0, 0), (0, 1)))
    wef = w_edge_feat.astype(f32)
    w_ef1 = jnp.pad(wef[:, :F - 1].T, ((0, 1), (0, 0)))
    w_ef2s = wef[:, F - 1].reshape(1, F) / maxn
    w_init_t = w_node_init.T.astype(f32)
    w_msg = jnp.stack([w.T for w in (w_msg_0, w_msg_1, w_msg_2)]).astype(f32)
    w_upd = jnp.stack([w.T for w in (w_upd_0, w_upd_1, w_upd_2)]).astype(f32)
    w_pool_t = w_pool.T.astype(f32)
    w_r = w_read.reshape(2 * F).astype(f32)
    w_rp = w_r[:F].reshape(1, F)
    w_rl = w_r[F:].reshape(1, F)
    b_r = b_read.reshape(1, 1).astype(f32)

    TB = 16
    while B % TB:
        TB //= 2
    L = 3

    def full(shape):
        nd = len(shape)
        return pl.BlockSpec(shape, lambda i, _nd=nd: (0,) * _nd)

    out = pl.pallas_call(
        _mpnn_kernel,
        out_shape=jax.ShapeDtypeStruct((B, 1, V), f32),
        grid=(B // TB,),
        in_specs=[
            pl.BlockSpec((TB, 2 * V, V), lambda i: (i, 0, 0)),
            full((n_obs, F)), full((1, F)), full((n_obs, F)),
            full((F, F)), full((1, F)),
            full((L, 2 * F, F)), full((L, 2 * F, F)),
            full((F, F)), full((1, F)), full((1, F)), full((1, 1)),
        ],
        out_specs=pl.BlockSpec((TB, 1, V), lambda i: (i, 0, 0)),
        compiler_params=pltpu.CompilerParams(
            dimension_semantics=("parallel",)),
    )(obs,
      w_init_t, we_a, we_n, w_ef1, w_ef2s,
      w_msg, w_upd,
      w_pool_t, w_rp, w_rl, b_r)

    return jnp.squeeze(out)
